# baseline (speedup 1.0000x reference)
import functools

import jax
import jax.numpy as jnp
from jax import lax
from jax.experimental import pallas as pl
from jax.experimental.pallas import tpu as pltpu
from jax.experimental.pallas import tpu_sc as plsc

D = 64
DP = 128
V = 1000000
VP = 1000064
N0 = 4096
N1 = 50
NC, NS = 2, 16
NW = NC * NS
CHUNK = 128
TCH = 7812
PAIRS = TCH // NW // 2

_mesh = plsc.VectorSubcoreMesh(core_axis_name="c", subcore_axis_name="s")


@functools.partial(
    pl.kernel,
    mesh=_mesh,
    out_type=jax.ShapeDtypeStruct((VP, DP), jnp.float32),
    scratch_types=[
        pltpu.VMEM((D, CHUNK), jnp.float32),
        pltpu.VMEM((D, CHUNK), jnp.float32),
        pltpu.VMEM((CHUNK, DP), jnp.float32),
        pltpu.VMEM((CHUNK, DP), jnp.float32),
        pltpu.SemaphoreType.DMA,
        pltpu.SemaphoreType.DMA,
        pltpu.SemaphoreType.DMA,
        pltpu.SemaphoreType.DMA,
    ],
    compiler_params=pltpu.CompilerParams(needs_layout_passes=False),
)
def _transpose_kernel(tab_hbm, tail_hbm, tpx_hbm, gb0, gb1, ob0, ob1,
                      r0, r1, w0, w1):
    wid = lax.axis_index("s") * NC + lax.axis_index("c")
    lane = lax.iota(jnp.int32, 16)
    rows = [lane + k * 16 for k in range(4)]

    def fire_read(c, gb, rsem):
        for db in range(8):
            pltpu.async_copy(
                tab_hbm.at[pl.ds(db * 8, 8), pl.ds(c * CHUNK, CHUNK)],
                gb.at[pl.ds(db * 8, 8)],
                rsem,
            )

    def wait_read(gb, rsem):
        for db in range(8):
            pltpu.make_async_copy(
                tab_hbm.at[pl.ds(0, 8), pl.ds(0, CHUNK)],
                gb.at[pl.ds(db * 8, 8)],
                rsem,
            ).wait()

    def transpose(gb, ob):
        @plsc.parallel_loop(0, CHUNK, 1, unroll=4)
        def body(l):
            col = jnp.broadcast_to(l, (16,)).astype(jnp.int32)
            for k in range(4):
                v = plsc.load_gather(gb, [rows[k], col])
                ob[l, pl.ds(k * 16, 16)] = v

    def write(c, ob, wsem):
        pltpu.async_copy(ob, tpx_hbm.at[pl.ds(c * CHUNK, CHUNK)], wsem)

    def drain_write(ob, wsem):
        pltpu.make_async_copy(ob, tpx_hbm.at[pl.ds(0, CHUNK)], wsem).wait()

    fire_read(wid, gb0, r0)

    def body(p, carry):
        fire_read(wid + NW + 2 * NW * p, gb1, r1)
        wait_read(gb0, r0)

        @pl.when(p > 0)
        def _():
            drain_write(ob0, w0)

        transpose(gb0, ob0)
        write(wid + 2 * NW * p, ob0, w0)

        @pl.when(p < PAIRS - 1)
        def _():
            fire_read(wid + 2 * NW * (p + 1), gb0, r0)

        wait_read(gb1, r1)

        @pl.when(p > 0)
        def _():
            drain_write(ob1, w1)

        transpose(gb1, ob1)
        write(wid + NW + 2 * NW * p, ob1, w1)
        return carry

    lax.fori_loop(0, PAIRS, body, 0)
    drain_write(ob0, w0)
    drain_write(ob1, w1)

    @pl.when(wid < TCH - NW * 2 * PAIRS)
    def _():
        c = 2 * NW * PAIRS + wid
        fire_read(c, gb0, r0)
        wait_read(gb0, r0)
        transpose(gb0, ob0)
        write(c, ob0, w0)
        drain_write(ob0, w0)

    @pl.when(wid == NW - 1)
    def _():
        pltpu.sync_copy(tail_hbm, ob1.at[pl.ds(0, D)])
        pltpu.sync_copy(ob1.at[pl.ds(0, D)], tpx_hbm.at[pl.ds(V - D, D)])


@functools.partial(
    pl.kernel,
    mesh=_mesh,
    out_type=jax.ShapeDtypeStruct((N1, D, N0), jnp.float32),
    scratch_types=[
        pltpu.VMEM((N1, CHUNK), jnp.int32),
        pltpu.VMEM((CHUNK, DP), jnp.float32),
        pltpu.VMEM((CHUNK, DP), jnp.float32),
        pltpu.VMEM((8, 8, CHUNK), jnp.float32),
        pltpu.VMEM((8, 8, CHUNK), jnp.float32),
        pltpu.SemaphoreType.DMA,
        pltpu.SemaphoreType.DMA,
        pltpu.SemaphoreType.DMA,
        pltpu.SemaphoreType.DMA,
    ],
    compiler_params=pltpu.CompilerParams(needs_layout_passes=False),
)
def _gather_kernel(idx_hbm, table_hbm, out_hbm, idx_v, buf0, buf1,
                   obuf0, obuf1, g0, g1, o0, o1):
    nc = lax.axis_index("s") * NC + lax.axis_index("c")
    pltpu.sync_copy(idx_hbm.at[:, nc], idx_v)

    lane = lax.iota(jnp.int32, 16)
    rows = [lane + lb * 16 for lb in range(8)]

    def fire(n1, buf, gsem):
        pltpu.async_copy(table_hbm.at[idx_v.at[n1]], buf, gsem)

    def wait_gather(buf, gsem):
        pltpu.make_async_copy(table_hbm.at[idx_v.at[0]], buf, gsem).wait()

    def transpose_tile(buf, obuf):
        obuf2 = obuf.reshape(D, CHUNK)

        @plsc.parallel_loop(0, D, 1, unroll=4)
        def body(d):
            col = jnp.broadcast_to(d, (16,)).astype(jnp.int32)
            for lb in range(8):
                v = plsc.load_gather(buf, [rows[lb], col])
                obuf2[d, pl.ds(lb * 16, 16)] = v

    def flush_tile(n1, obuf, osem):
        for db in range(8):
            pltpu.async_copy(
                obuf.at[db],
                out_hbm.at[n1, pl.ds(db * 8, 8), pl.ds(nc * CHUNK, CHUNK)],
                osem,
            )

    def drain_flush(obuf, osem):
        for db in range(8):
            pltpu.make_async_copy(
                obuf.at[db],
                out_hbm.at[0, pl.ds(db * 8, 8), pl.ds(nc * CHUNK, CHUNK)],
                osem,
            ).wait()

    fire(0, buf0, g0)

    def body(p, carry):
        a = 2 * p
        fire(a + 1, buf1, g1)
        wait_gather(buf0, g0)

        @pl.when(p > 0)
        def _():
            drain_flush(obuf0, o0)

        transpose_tile(buf0, obuf0)
        flush_tile(a, obuf0, o0)

        @pl.when(p < N1 // 2 - 1)
        def _():
            fire(a + 2, buf0, g0)

        wait_gather(buf1, g1)

        @pl.when(p > 0)
        def _():
            drain_flush(obuf1, o1)

        transpose_tile(buf1, obuf1)
        flush_tile(a + 1, obuf1, o1)
        return carry

    lax.fori_loop(0, N1 // 2, body, 0)
    drain_flush(obuf0, o0)
    drain_flush(obuf1, o1)


def kernel(words, table):
    idx = words.T.reshape(N1, NW, CHUNK)
    tail = jnp.pad(table[V - D:], ((0, 0), (0, DP - D)))
    tpx = _transpose_kernel(table.T, tail)
    out3d = _gather_kernel(idx, tpx)
    return out3d.transpose(2, 0, 1)


# speedup vs baseline: 1.3793x; 1.3793x over previous
import functools

import jax
import jax.numpy as jnp
from jax import lax
from jax.experimental import pallas as pl
from jax.experimental.pallas import tpu as pltpu
from jax.experimental.pallas import tpu_sc as plsc

D = 64
DP = 128
N0 = 4096
N1 = 50
NC, NS = 2, 16
NW = NC * NS
CHUNK = 128

_mesh = plsc.VectorSubcoreMesh(core_axis_name="c", subcore_axis_name="s")


@functools.partial(
    pl.kernel,
    mesh=_mesh,
    out_type=jax.ShapeDtypeStruct((N1, D, N0), jnp.float32),
    scratch_types=[
        pltpu.VMEM((N1, CHUNK), jnp.int32),
        pltpu.VMEM((CHUNK, DP), jnp.float32),
        pltpu.VMEM((CHUNK, DP), jnp.float32),
        pltpu.VMEM((8, 8, CHUNK), jnp.float32),
        pltpu.VMEM((8, 8, CHUNK), jnp.float32),
        pltpu.SemaphoreType.DMA,
        pltpu.SemaphoreType.DMA,
        pltpu.SemaphoreType.DMA,
        pltpu.SemaphoreType.DMA,
    ],
    compiler_params=pltpu.CompilerParams(needs_layout_passes=False),
)
def _gather_kernel(idx_hbm, table_hbm, out_hbm, idx_v, buf0, buf1,
                   obuf0, obuf1, g0, g1, o0, o1):
    nc = lax.axis_index("s") * NC + lax.axis_index("c")
    pltpu.sync_copy(idx_hbm.at[:, nc], idx_v)

    lane = lax.iota(jnp.int32, 16)
    rows = [lane + lb * 16 for lb in range(8)]

    def fire(n1, buf, gsem):
        pltpu.async_copy(table_hbm.at[idx_v.at[n1]], buf, gsem)

    def wait_gather(buf, gsem):
        pltpu.make_async_copy(table_hbm.at[idx_v.at[0]], buf, gsem).wait()

    def transpose_tile(buf, obuf):
        obuf2 = obuf.reshape(D, CHUNK)

        @plsc.parallel_loop(0, D, 1, unroll=4)
        def body(d):
            col = jnp.broadcast_to(d, (16,)).astype(jnp.int32)
            for lb in range(8):
                v = plsc.load_gather(buf, [rows[lb], col])
                obuf2[d, pl.ds(lb * 16, 16)] = v

    def flush_tile(n1, obuf, osem):
        for db in range(8):
            pltpu.async_copy(
                obuf.at[db],
                out_hbm.at[n1, pl.ds(db * 8, 8), pl.ds(nc * CHUNK, CHUNK)],
                osem,
            )

    def drain_flush(obuf, osem):
        for db in range(8):
            pltpu.make_async_copy(
                obuf.at[db],
                out_hbm.at[0, pl.ds(db * 8, 8), pl.ds(nc * CHUNK, CHUNK)],
                osem,
            ).wait()

    fire(0, buf0, g0)

    def body(p, carry):
        a = 2 * p
        fire(a + 1, buf1, g1)
        wait_gather(buf0, g0)

        @pl.when(p > 0)
        def _():
            drain_flush(obuf0, o0)

        transpose_tile(buf0, obuf0)
        flush_tile(a, obuf0, o0)

        @pl.when(p < N1 // 2 - 1)
        def _():
            fire(a + 2, buf0, g0)

        wait_gather(buf1, g1)

        @pl.when(p > 0)
        def _():
            drain_flush(obuf1, o1)

        transpose_tile(buf1, obuf1)
        flush_tile(a + 1, obuf1, o1)
        return carry

    lax.fori_loop(0, N1 // 2, body, 0)
    drain_flush(obuf0, o0)
    drain_flush(obuf1, o1)


def kernel(words, table):
    idx = words.T.reshape(N1, NW, CHUNK)
    tpad = jnp.pad(table, ((0, 0), (0, DP - D)))
    out3d = _gather_kernel(idx, tpad)
    return out3d.transpose(2, 0, 1)
